# Initial kernel scaffold; baseline (speedup 1.0000x reference)
import jax
import jax.numpy as jnp
from jax.experimental import pallas as pl


def kernel(input_fmap, theta):
    raise NotImplementedError("write your pallas kernel here")


# speedup vs baseline: 1.7635x; 1.7635x over previous
import functools

import jax
import jax.numpy as jnp
from jax import lax
from jax.experimental import pallas as pl
from jax.experimental.pallas import tpu as pltpu
from jax.experimental.pallas import tpu_sc as plsc

B, H, W, C = 4, 384, 384, 96
N = B * H * W
NC, NS, L = 2, 16, 16
NW = NC * NS
PER_W = N // NW
G = 128
NGROUPS = PER_W // G
HW = H * W


_GDN = lax.GatherDimensionNumbers(
    offset_dims=(), collapsed_slice_dims=(0,), start_index_map=(0,))


def _lane_bcast(vec, kk):
    sel = jnp.full((L, 1), kk, dtype=jnp.int32)
    return lax.gather(vec, sel, _GDN, (1,),
                      mode=lax.GatherScatterMode.PROMISE_IN_BOUNDS)


def _floorclip(v):
    vc = jnp.clip(v, -1.0, 385.0)
    t = vc.astype(jnp.int32)
    tf = t.astype(jnp.float32)
    f0 = t - jnp.where(tf > vc, 1, 0)
    i0 = jnp.clip(f0, 0, W - 1)
    i1 = jnp.clip(f0 + 1, 0, W - 1)
    return i0, i1


def _sc_sample(table, xs, ys):
    mesh = plsc.VectorSubcoreMesh(core_axis_name="c", subcore_axis_name="s")

    @functools.partial(
        pl.kernel,
        mesh=mesh,
        out_type=jax.ShapeDtypeStruct((N, C), jnp.float32),
        compiler_params=pltpu.CompilerParams(use_tc_tiling_on_sc=False),
        scratch_types=[
            pltpu.VMEM((G,), jnp.float32),
            pltpu.VMEM((G,), jnp.float32),
            pltpu.VMEM((G,), jnp.int32),
            pltpu.VMEM((G,), jnp.int32),
            pltpu.VMEM((G,), jnp.int32),
            pltpu.VMEM((G,), jnp.int32),
            pltpu.VMEM((G,), jnp.float32),
            pltpu.VMEM((G,), jnp.float32),
            pltpu.VMEM((G,), jnp.float32),
            pltpu.VMEM((G,), jnp.float32),
            pltpu.VMEM((G, C), jnp.float32),
            pltpu.VMEM((G, C), jnp.float32),
            pltpu.VMEM((G, C), jnp.float32),
            pltpu.VMEM((G, C), jnp.float32),
            pltpu.VMEM((G, C), jnp.float32),
            pltpu.SemaphoreType.DMA,
            pltpu.SemaphoreType.DMA,
            pltpu.SemaphoreType.DMA,
            pltpu.SemaphoreType.DMA,
        ],
    )
    def k(table_hbm, xs_hbm, ys_hbm, out_hbm,
          xs_v, ys_v, ia_v, ib_v, ic_v, id_v, wa_v, wb_v, wc_v, wd_v,
          ra_v, rb_v, rc_v, rd_v, out_v, sem_a, sem_b, sem_c, sem_d):
        wid = lax.axis_index("s") * NC + lax.axis_index("c")
        wbase = wid * PER_W
        tbase = (wbase // HW) * HW

        def group_body(g, _):
            gbase = wbase + g * G
            pltpu.sync_copy(xs_hbm.at[pl.ds(gbase, G)], xs_v)
            pltpu.sync_copy(ys_hbm.at[pl.ds(gbase, G)], ys_v)

            for i in range(G // L):
                sl = pl.ds(i * L, L)
                x = xs_v[sl]
                y = ys_v[sl]
                x0, x1 = _floorclip(x)
                y0, y1 = _floorclip(y)
                x0f = x0.astype(jnp.float32)
                x1f = x1.astype(jnp.float32)
                y0f = y0.astype(jnp.float32)
                y1f = y1.astype(jnp.float32)
                wa_v[sl] = (x1f - x) * (y1f - y)
                wb_v[sl] = (x1f - x) * (y - y0f)
                wc_v[sl] = (x - x0f) * (y1f - y)
                wd_v[sl] = (x - x0f) * (y - y0f)
                r0 = tbase + y0 * W
                r1 = tbase + y1 * W
                ia_v[sl] = r0 + x0
                ib_v[sl] = r1 + x0
                ic_v[sl] = r0 + x1
                id_v[sl] = r1 + x1

            ha = pltpu.async_copy(table_hbm.at[ia_v], ra_v, sem_a)
            hb = pltpu.async_copy(table_hbm.at[ib_v], rb_v, sem_b)
            hc = pltpu.async_copy(table_hbm.at[ic_v], rc_v, sem_c)
            hd = pltpu.async_copy(table_hbm.at[id_v], rd_v, sem_d)
            ha.wait()
            hb.wait()
            hc.wait()
            hd.wait()

            def blend16(i, _):
                base16 = i * L
                wa16 = wa_v[pl.ds(base16, L)]
                wb16 = wb_v[pl.ds(base16, L)]
                wc16 = wc_v[pl.ds(base16, L)]
                wd16 = wd_v[pl.ds(base16, L)]
                for kk in range(L):
                    p = base16 + kk
                    was = _lane_bcast(wa16, kk)
                    wbs = _lane_bcast(wb16, kk)
                    wcs = _lane_bcast(wc16, kk)
                    wds = _lane_bcast(wd16, kk)
                    for c in range(C // L):
                        cs = pl.ds(c * L, L)
                        pa = was * ra_v[p, cs]
                        pb = wbs * rb_v[p, cs]
                        pc = wcs * rc_v[p, cs]
                        pd = wds * rd_v[p, cs]
                        out_v[p, cs] = ((pa + pb) + pc) + pd
                return 0

            lax.fori_loop(0, G // L, blend16, 0)
            pltpu.sync_copy(out_v, out_hbm.at[pl.ds(gbase, G)])
            return 0

        lax.fori_loop(0, NGROUPS, group_body, 0)

    return k(table, xs, ys)


def kernel(input_fmap, theta):
    theta3 = theta.reshape(B, 3, 3).astype(jnp.float32)
    x = jnp.linspace(0.0, 256.0, W)
    y = jnp.linspace(0.0, 256.0, H)
    x_t, y_t = jnp.meshgrid(x, y)
    ones = jnp.ones_like(x_t.reshape(-1))
    sampling_grid = jnp.stack([x_t.reshape(-1), y_t.reshape(-1), ones])
    sampling_grid = jnp.tile(sampling_grid[None, :, :], (B, 1, 1))
    sampling_grid = sampling_grid.astype(jnp.float32)
    batch_grids = jnp.matmul(theta3, sampling_grid)
    xs = (batch_grids[:, 0, :] / batch_grids[:, 2, :]).reshape(-1)
    ys = (batch_grids[:, 1, :] / batch_grids[:, 2, :]).reshape(-1)

    table = input_fmap.reshape(N, C)
    out = _sc_sample(table, xs, ys)
    return out.reshape(B, H, W, C)


# speedup vs baseline: 3.1520x; 1.7874x over previous
import functools

import jax
import jax.numpy as jnp
from jax import lax
from jax.experimental import pallas as pl
from jax.experimental.pallas import tpu as pltpu
from jax.experimental.pallas import tpu_sc as plsc

B, H, W, C = 4, 384, 384, 96
N = B * H * W
NC, NS, L = 2, 16, 16
NW = NC * NS
G = 128
NG = N // G
TPW = NG // NW
GPB = (H * W) // G
BY, BX = 4, 16
NV = G // L

_GDN = lax.GatherDimensionNumbers(
    offset_dims=(), collapsed_slice_dims=(0,), start_index_map=(0,))


def _lane_bcast(vec, kk):
    sel = jnp.full((L, 1), kk, dtype=jnp.int32)
    return lax.gather(vec, sel, _GDN, (1,),
                      mode=lax.GatherScatterMode.PROMISE_IN_BOUNDS)


def _floorclip(v):
    vc = jnp.clip(v, -1.0, 385.0)
    t = vc.astype(jnp.int32)
    tf = t.astype(jnp.float32)
    f0 = t - jnp.where(tf > vc, 1, 0)
    i0 = jnp.clip(f0, 0, W - 1)
    i1 = jnp.clip(f0 + 1, 0, W - 1)
    return i0, i1


def _sc_sample(table, xs, ys):
    mesh = plsc.VectorSubcoreMesh(core_axis_name="c", subcore_axis_name="s")

    def slot_scratch():
        return [
            pltpu.VMEM((G,), jnp.float32),
            pltpu.VMEM((G,), jnp.float32),
            pltpu.VMEM((G,), jnp.int32),
            pltpu.VMEM((G,), jnp.int32),
            pltpu.VMEM((G,), jnp.int32),
            pltpu.VMEM((G,), jnp.int32),
            pltpu.VMEM((G,), jnp.float32),
            pltpu.VMEM((G,), jnp.float32),
            pltpu.VMEM((G,), jnp.float32),
            pltpu.VMEM((G,), jnp.float32),
            pltpu.VMEM((G,), jnp.int32),
            pltpu.VMEM((G,), jnp.int32),
            pltpu.VMEM((G,), jnp.int32),
            pltpu.VMEM((G,), jnp.int32),
            pltpu.VMEM((G, C), jnp.float32),
            pltpu.VMEM((G, C), jnp.float32),
            pltpu.VMEM((G, C), jnp.float32),
            pltpu.VMEM((G, C), jnp.float32),
            pltpu.SemaphoreType.DMA,
            pltpu.SemaphoreType.DMA,
        ]

    @functools.partial(
        pl.kernel,
        mesh=mesh,
        out_type=jax.ShapeDtypeStruct((N, C), jnp.float32),
        compiler_params=pltpu.CompilerParams(use_tc_tiling_on_sc=False,
                                             needs_layout_passes=False),
        scratch_types=slot_scratch() + slot_scratch() + [
            pltpu.VMEM((G, C), jnp.float32),
            pltpu.SemaphoreType.DMA,
        ],
    )
    def k(table_hbm, xs_hbm, ys_hbm, out_hbm, *scr):
        slots = (scr[:20], scr[20:40])
        out_v, sem_o = scr[40], scr[41]
        wid = lax.axis_index("s") * NC + lax.axis_index("c")

        def gid_of(t):
            return wid + t * NW

        def fire_xy(t, s):
            xs_v, ys_v, sem_x = s[0], s[1], s[18]
            gb = gid_of(t) * G
            pltpu.async_copy(xs_hbm.at[pl.ds(gb, G)], xs_v, sem_x)
            pltpu.async_copy(ys_hbm.at[pl.ds(gb, G)], ys_v, sem_x)

        def minmax(x0_v, x1_v, y0_v, y1_v):
            xmn = x0_v[pl.ds(0, L)]
            xmx = x1_v[pl.ds(0, L)]
            ymn = y0_v[pl.ds(0, L)]
            ymx = y1_v[pl.ds(0, L)]
            for i in range(1, NV):
                sl = pl.ds(i * L, L)
                xmn = jnp.minimum(xmn, x0_v[sl])
                xmx = jnp.maximum(xmx, x1_v[sl])
                ymn = jnp.minimum(ymn, y0_v[sl])
                ymx = jnp.maximum(ymx, y1_v[sl])
            xmn = lax.reduce_min(xmn, (0,))
            xmx = lax.reduce_max(xmx, (0,))
            ymn = lax.reduce_min(ymn, (0,))
            ymx = lax.reduce_max(ymx, (0,))
            isbox = jnp.logical_and(xmx - xmn < BX, ymx - ymn < BY)
            ymn_c = jnp.minimum(ymn, H - BY)
            xmn_c = jnp.minimum(xmn, W - BX)
            return isbox, ymn_c, xmn_c

        def phase1(t, s):
            (xs_v, ys_v, x0_v, x1_v, y0_v, y1_v, wa_v, wb_v, wc_v, wd_v,
             ja_v, jb_v, jc_v, jd_v, ra_v, rb_v, rc_v, rd_v,
             sem_x, sem_f) = s
            gid = gid_of(t)
            tbase = (gid // GPB) * (H * W)
            pltpu.make_async_copy(xs_hbm.at[pl.ds(0, G)], xs_v, sem_x).wait()
            pltpu.make_async_copy(ys_hbm.at[pl.ds(0, G)], ys_v, sem_x).wait()
            res = []
            for i in range(NV):
                sl = pl.ds(i * L, L)
                x = xs_v[sl]
                y = ys_v[sl]
                x0, x1 = _floorclip(x)
                y0, y1 = _floorclip(y)
                x0f = x0.astype(jnp.float32)
                x1f = x1.astype(jnp.float32)
                y0f = y0.astype(jnp.float32)
                y1f = y1.astype(jnp.float32)
                res.append((x0, x1, y0, y1,
                            (x1f - x) * (y1f - y), (x1f - x) * (y - y0f),
                            (x - x0f) * (y1f - y), (x - x0f) * (y - y0f)))
            for i in range(NV):
                sl = pl.ds(i * L, L)
                x0, x1, y0, y1, wa, wb, wc, wd = res[i]
                x0_v[sl] = x0
                x1_v[sl] = x1
                y0_v[sl] = y0
                y1_v[sl] = y1
                wa_v[sl] = wa
                wb_v[sl] = wb
                wc_v[sl] = wc
                wd_v[sl] = wd
            @pl.when(t <= TPW - 3)
            def _():
                fire_xy(t + 2, s)

            isbox, ymn_c, xmn_c = minmax(x0_v, x1_v, y0_v, y1_v)

            @pl.when(isbox)
            def _():
                for i in range(NV):
                    sl = pl.ds(i * L, L)
                    dx0 = x0_v[sl] - xmn_c
                    dx1 = x1_v[sl] - xmn_c
                    ry0 = (y0_v[sl] - ymn_c) * BX
                    ry1 = (y1_v[sl] - ymn_c) * BX
                    ja_v[sl] = ry0 + dx0
                    jb_v[sl] = ry1 + dx0
                    jc_v[sl] = ry0 + dx1
                    jd_v[sl] = ry1 + dx1
                rowbase = tbase + ymn_c * W + xmn_c
                for r in range(BY):
                    pltpu.async_copy(
                        table_hbm.at[pl.ds(rowbase + r * W, BX)],
                        rc_v.at[pl.ds(r * BX, BX), :], sem_f)

            @pl.when(jnp.logical_not(isbox))
            def _():
                for i in range(NV):
                    sl = pl.ds(i * L, L)
                    r0 = tbase + y0_v[sl] * W
                    r1 = tbase + y1_v[sl] * W
                    ja_v[sl] = r0 + x0_v[sl]
                    jb_v[sl] = r1 + x0_v[sl]
                    jc_v[sl] = r0 + x1_v[sl]
                    jd_v[sl] = r1 + x1_v[sl]
                pltpu.async_copy(table_hbm.at[ja_v], ra_v, sem_f)
                pltpu.async_copy(table_hbm.at[jb_v], rb_v, sem_f)
                pltpu.async_copy(table_hbm.at[jc_v], rc_v, sem_f)
                pltpu.async_copy(table_hbm.at[jd_v], rd_v, sem_f)

        def blend(t, s, first):
            (xs_v, ys_v, x0_v, x1_v, y0_v, y1_v, wa_v, wb_v, wc_v, wd_v,
             ja_v, jb_v, jc_v, jd_v, ra_v, rb_v, rc_v, rd_v,
             sem_x, sem_f) = s
            gid = gid_of(t)
            isbox, _ymn, _xmn = minmax(x0_v, x1_v, y0_v, y1_v)

            @pl.when(jnp.logical_not(first))
            def _():
                pltpu.make_async_copy(out_v, out_hbm.at[pl.ds(0, G)],
                                      sem_o).wait()

            cols = [jnp.arange(L, dtype=jnp.int32) + c * L
                    for c in range(C // L)]

            @pl.when(isbox)
            def _():
                for r in range(BY):
                    pltpu.make_async_copy(
                        table_hbm.at[pl.ds(0, BX)],
                        rc_v.at[pl.ds(r * BX, BX), :], sem_f).wait()

                def blend16(i, _):
                    sl = pl.ds(i * L, L)
                    wa16 = wa_v[sl]
                    wb16 = wb_v[sl]
                    wc16 = wc_v[sl]
                    wd16 = wd_v[sl]
                    ja16 = ja_v[sl]
                    jb16 = jb_v[sl]
                    jc16 = jc_v[sl]
                    jd16 = jd_v[sl]
                    for kk in range(L):
                        p = i * L + kk
                        was = _lane_bcast(wa16, kk)
                        wbs = _lane_bcast(wb16, kk)
                        wcs = _lane_bcast(wc16, kk)
                        wds = _lane_bcast(wd16, kk)
                        jas = _lane_bcast(ja16, kk)
                        jbs = _lane_bcast(jb16, kk)
                        jcs = _lane_bcast(jc16, kk)
                        jds = _lane_bcast(jd16, kk)
                        for c in range(C // L):
                            va = plsc.load_gather(rc_v, [jas, cols[c]])
                            vb = plsc.load_gather(rc_v, [jbs, cols[c]])
                            vc = plsc.load_gather(rc_v, [jcs, cols[c]])
                            vd = plsc.load_gather(rc_v, [jds, cols[c]])
                            pa = was * va
                            pb = wbs * vb
                            pc = wcs * vc
                            pd = wds * vd
                            out_v[p, pl.ds(c * L, L)] = ((pa + pb) + pc) + pd
                    return 0

                lax.fori_loop(0, NV, blend16, 0)

            @pl.when(jnp.logical_not(isbox))
            def _():
                pltpu.make_async_copy(table_hbm.at[ja_v], ra_v, sem_f).wait()
                pltpu.make_async_copy(table_hbm.at[jb_v], rb_v, sem_f).wait()
                pltpu.make_async_copy(table_hbm.at[jc_v], rc_v, sem_f).wait()
                pltpu.make_async_copy(table_hbm.at[jd_v], rd_v, sem_f).wait()

                def blend16(i, _):
                    sl = pl.ds(i * L, L)
                    wa16 = wa_v[sl]
                    wb16 = wb_v[sl]
                    wc16 = wc_v[sl]
                    wd16 = wd_v[sl]
                    for kk in range(L):
                        p = i * L + kk
                        was = _lane_bcast(wa16, kk)
                        wbs = _lane_bcast(wb16, kk)
                        wcs = _lane_bcast(wc16, kk)
                        wds = _lane_bcast(wd16, kk)
                        for c in range(C // L):
                            cs = pl.ds(c * L, L)
                            pa = was * ra_v[p, cs]
                            pb = wbs * rb_v[p, cs]
                            pc = wcs * rc_v[p, cs]
                            pd = wds * rd_v[p, cs]
                            out_v[p, cs] = ((pa + pb) + pc) + pd
                    return 0

                lax.fori_loop(0, NV, blend16, 0)

            pltpu.async_copy(out_v, out_hbm.at[pl.ds(gid * G, G)], sem_o)

        fire_xy(0, slots[0])
        fire_xy(1, slots[1])

        def pair_body(i, _):
            t0 = 2 * i
            phase1(t0, slots[0])

            @pl.when(i >= 1)
            def _():
                blend(t0 - 1, slots[1], jnp.bool_(False))

            phase1(t0 + 1, slots[1])
            blend(t0, slots[0], i == 0)
            return 0

        lax.fori_loop(0, TPW // 2, pair_body, 0)
        blend(TPW - 1, slots[1], jnp.bool_(False))
        pltpu.make_async_copy(out_v, out_hbm.at[pl.ds(0, G)], sem_o).wait()

    return k(table, xs, ys)


def kernel(input_fmap, theta):
    theta3 = theta.reshape(B, 3, 3).astype(jnp.float32)
    x = jnp.linspace(0.0, 256.0, W)
    y = jnp.linspace(0.0, 256.0, H)
    x_t, y_t = jnp.meshgrid(x, y)
    ones = jnp.ones_like(x_t.reshape(-1))
    sampling_grid = jnp.stack([x_t.reshape(-1), y_t.reshape(-1), ones])
    sampling_grid = jnp.tile(sampling_grid[None, :, :], (B, 1, 1))
    sampling_grid = sampling_grid.astype(jnp.float32)
    batch_grids = jnp.matmul(theta3, sampling_grid)
    xs = (batch_grids[:, 0, :] / batch_grids[:, 2, :]).reshape(-1)
    ys = (batch_grids[:, 1, :] / batch_grids[:, 2, :]).reshape(-1)

    table = input_fmap.reshape(N, C)
    out = _sc_sample(table, xs, ys)
    return out.reshape(B, H, W, C)


# speedup vs baseline: 3.8608x; 1.2249x over previous
import functools

import jax
import jax.numpy as jnp
from jax import lax
from jax.experimental import pallas as pl
from jax.experimental.pallas import tpu as pltpu
from jax.experimental.pallas import tpu_sc as plsc

B, H, W, C = 4, 384, 384, 96
N = B * H * W
NC, NS, L = 2, 16, 16
NW = NC * NS
G = 128
NG = N // G
TPW = NG // NW
GPB = (H * W) // G
BY, BX = 4, 16
TIERS = ((4, 16), (8, 8), (16, 4))
NV = G // L

_GDN = lax.GatherDimensionNumbers(
    offset_dims=(), collapsed_slice_dims=(0,), start_index_map=(0,))


def _lane_bcast(vec, kk):
    sel = jnp.full((L, 1), kk, dtype=jnp.int32)
    return lax.gather(vec, sel, _GDN, (1,),
                      mode=lax.GatherScatterMode.PROMISE_IN_BOUNDS)


def _floorclip(v):
    vc = jnp.clip(v, -1.0, 385.0)
    t = vc.astype(jnp.int32)
    tf = t.astype(jnp.float32)
    f0 = t - jnp.where(tf > vc, 1, 0)
    i0 = jnp.clip(f0, 0, W - 1)
    i1 = jnp.clip(f0 + 1, 0, W - 1)
    return i0, i1


def _sc_sample(table, xs, ys):
    mesh = plsc.VectorSubcoreMesh(core_axis_name="c", subcore_axis_name="s")

    def slot_scratch():
        return [
            pltpu.VMEM((G,), jnp.float32),
            pltpu.VMEM((G,), jnp.float32),
            pltpu.VMEM((G,), jnp.int32),
            pltpu.VMEM((G,), jnp.int32),
            pltpu.VMEM((G,), jnp.int32),
            pltpu.VMEM((G,), jnp.int32),
            pltpu.VMEM((G,), jnp.float32),
            pltpu.VMEM((G,), jnp.float32),
            pltpu.VMEM((G,), jnp.float32),
            pltpu.VMEM((G,), jnp.float32),
            pltpu.VMEM((G,), jnp.int32),
            pltpu.VMEM((G,), jnp.int32),
            pltpu.VMEM((G,), jnp.int32),
            pltpu.VMEM((G,), jnp.int32),
            pltpu.VMEM((G, C), jnp.float32),
            pltpu.VMEM((G, C), jnp.float32),
            pltpu.VMEM((G, C), jnp.float32),
            pltpu.VMEM((G, C), jnp.float32),
            pltpu.SemaphoreType.DMA,
            pltpu.SemaphoreType.DMA,
        ]

    @functools.partial(
        pl.kernel,
        mesh=mesh,
        out_type=jax.ShapeDtypeStruct((N, C), jnp.float32),
        compiler_params=pltpu.CompilerParams(use_tc_tiling_on_sc=False,
                                             needs_layout_passes=False),
        scratch_types=slot_scratch() + slot_scratch() + [
            pltpu.VMEM((G, C), jnp.float32),
            pltpu.SemaphoreType.DMA,
        ],
    )
    def k(table_hbm, xs_hbm, ys_hbm, out_hbm, *scr):
        slots = (scr[:20], scr[20:40])
        out_v, sem_o = scr[40], scr[41]
        wid = lax.axis_index("s") * NC + lax.axis_index("c")

        def gid_of(t):
            return wid + t * NW

        def fire_xy(t, s):
            xs_v, ys_v, sem_x = s[0], s[1], s[18]
            gb = gid_of(t) * G
            pltpu.async_copy(xs_hbm.at[pl.ds(gb, G)], xs_v, sem_x)
            pltpu.async_copy(ys_hbm.at[pl.ds(gb, G)], ys_v, sem_x)

        def minmax(x0_v, x1_v, y0_v, y1_v):
            xmn = x0_v[pl.ds(0, L)]
            xmx = x1_v[pl.ds(0, L)]
            ymn = y0_v[pl.ds(0, L)]
            ymx = y1_v[pl.ds(0, L)]
            for i in range(1, NV):
                sl = pl.ds(i * L, L)
                xmn = jnp.minimum(xmn, x0_v[sl])
                xmx = jnp.maximum(xmx, x1_v[sl])
                ymn = jnp.minimum(ymn, y0_v[sl])
                ymx = jnp.maximum(ymx, y1_v[sl])
            xmn = lax.reduce_min(xmn, (0,))
            xmx = lax.reduce_max(xmx, (0,))
            ymn = lax.reduce_min(ymn, (0,))
            ymx = lax.reduce_max(ymx, (0,))
            sels = []
            nofit = None
            for byt, bxt in TIERS:
                fit = jnp.logical_and(xmx - xmn < bxt, ymx - ymn < byt)
                sel = fit if nofit is None else jnp.logical_and(nofit, fit)
                nf = jnp.logical_not(fit)
                nofit = nf if nofit is None else jnp.logical_and(nofit, nf)
                sels.append(sel)
            sels.append(nofit)
            return sels, ymn, xmn

        def phase1(t, s):
            (xs_v, ys_v, x0_v, x1_v, y0_v, y1_v, wa_v, wb_v, wc_v, wd_v,
             ja_v, jb_v, jc_v, jd_v, ra_v, rb_v, rc_v, rd_v,
             sem_x, sem_f) = s
            gid = gid_of(t)
            tbase = (gid // GPB) * (H * W)
            pltpu.make_async_copy(xs_hbm.at[pl.ds(0, G)], xs_v, sem_x).wait()
            pltpu.make_async_copy(ys_hbm.at[pl.ds(0, G)], ys_v, sem_x).wait()
            res = []
            for i in range(NV):
                sl = pl.ds(i * L, L)
                x = xs_v[sl]
                y = ys_v[sl]
                x0, x1 = _floorclip(x)
                y0, y1 = _floorclip(y)
                x0f = x0.astype(jnp.float32)
                x1f = x1.astype(jnp.float32)
                y0f = y0.astype(jnp.float32)
                y1f = y1.astype(jnp.float32)
                res.append((x0, x1, y0, y1,
                            (x1f - x) * (y1f - y), (x1f - x) * (y - y0f),
                            (x - x0f) * (y1f - y), (x - x0f) * (y - y0f)))
            for i in range(NV):
                sl = pl.ds(i * L, L)
                x0, x1, y0, y1, wa, wb, wc, wd = res[i]
                x0_v[sl] = x0
                x1_v[sl] = x1
                y0_v[sl] = y0
                y1_v[sl] = y1
                wa_v[sl] = wa
                wb_v[sl] = wb
                wc_v[sl] = wc
                wd_v[sl] = wd
            @pl.when(t <= TPW - 3)
            def _():
                fire_xy(t + 2, s)

            sels, ymn, xmn = minmax(x0_v, x1_v, y0_v, y1_v)

            for vi, (byt, bxt) in enumerate(TIERS):
                @pl.when(sels[vi])
                def _(byt=byt, bxt=bxt):
                    ymn_c = jnp.minimum(ymn, H - byt)
                    xmn_c = jnp.minimum(xmn, W - bxt)
                    for i in range(NV):
                        sl = pl.ds(i * L, L)
                        dx0 = x0_v[sl] - xmn_c
                        dx1 = x1_v[sl] - xmn_c
                        ry0 = (y0_v[sl] - ymn_c) * bxt
                        ry1 = (y1_v[sl] - ymn_c) * bxt
                        ja_v[sl] = ry0 + dx0
                        jb_v[sl] = ry1 + dx0
                        jc_v[sl] = ry0 + dx1
                        jd_v[sl] = ry1 + dx1
                    rowbase = tbase + ymn_c * W + xmn_c
                    for r in range(byt):
                        pltpu.async_copy(
                            table_hbm.at[pl.ds(rowbase + r * W, bxt)],
                            rc_v.at[pl.ds(r * bxt, bxt), :], sem_f)

            @pl.when(sels[-1])
            def _():
                for i in range(NV):
                    sl = pl.ds(i * L, L)
                    r0 = tbase + y0_v[sl] * W
                    r1 = tbase + y1_v[sl] * W
                    ja_v[sl] = r0 + x0_v[sl]
                    jb_v[sl] = r1 + x0_v[sl]
                    jc_v[sl] = r0 + x1_v[sl]
                    jd_v[sl] = r1 + x1_v[sl]
                pltpu.async_copy(table_hbm.at[ja_v], ra_v, sem_f)
                pltpu.async_copy(table_hbm.at[jb_v], rb_v, sem_f)
                pltpu.async_copy(table_hbm.at[jc_v], rc_v, sem_f)
                pltpu.async_copy(table_hbm.at[jd_v], rd_v, sem_f)

        def blend(t, s, first):
            (xs_v, ys_v, x0_v, x1_v, y0_v, y1_v, wa_v, wb_v, wc_v, wd_v,
             ja_v, jb_v, jc_v, jd_v, ra_v, rb_v, rc_v, rd_v,
             sem_x, sem_f) = s
            gid = gid_of(t)
            sels, _ymn, _xmn = minmax(x0_v, x1_v, y0_v, y1_v)
            isbox = jnp.logical_not(sels[-1])

            @pl.when(jnp.logical_not(first))
            def _():
                pltpu.make_async_copy(out_v, out_hbm.at[pl.ds(0, G)],
                                      sem_o).wait()

            cols = [jnp.arange(L, dtype=jnp.int32) + c * L
                    for c in range(C // L)]

            for vi, (byt, bxt) in enumerate(TIERS):
                @pl.when(sels[vi])
                def _(byt=byt, bxt=bxt):
                    for r in range(byt):
                        pltpu.make_async_copy(
                            table_hbm.at[pl.ds(0, bxt)],
                            rc_v.at[pl.ds(r * bxt, bxt), :], sem_f).wait()

            @pl.when(isbox)
            def _():
                def blend16(i, _):
                    sl = pl.ds(i * L, L)
                    wa16 = wa_v[sl]
                    wb16 = wb_v[sl]
                    wc16 = wc_v[sl]
                    wd16 = wd_v[sl]
                    ja16 = ja_v[sl]
                    jb16 = jb_v[sl]
                    jc16 = jc_v[sl]
                    jd16 = jd_v[sl]
                    for kk in range(L):
                        p = i * L + kk
                        was = _lane_bcast(wa16, kk)
                        wbs = _lane_bcast(wb16, kk)
                        wcs = _lane_bcast(wc16, kk)
                        wds = _lane_bcast(wd16, kk)
                        jas = _lane_bcast(ja16, kk)
                        jbs = _lane_bcast(jb16, kk)
                        jcs = _lane_bcast(jc16, kk)
                        jds = _lane_bcast(jd16, kk)
                        for c in range(C // L):
                            va = plsc.load_gather(rc_v, [jas, cols[c]])
                            vb = plsc.load_gather(rc_v, [jbs, cols[c]])
                            vc = plsc.load_gather(rc_v, [jcs, cols[c]])
                            vd = plsc.load_gather(rc_v, [jds, cols[c]])
                            pa = was * va
                            pb = wbs * vb
                            pc = wcs * vc
                            pd = wds * vd
                            out_v[p, pl.ds(c * L, L)] = ((pa + pb) + pc) + pd
                    return 0

                lax.fori_loop(0, NV, blend16, 0)

            @pl.when(sels[-1])
            def _():
                pltpu.make_async_copy(table_hbm.at[ja_v], ra_v, sem_f).wait()
                pltpu.make_async_copy(table_hbm.at[jb_v], rb_v, sem_f).wait()
                pltpu.make_async_copy(table_hbm.at[jc_v], rc_v, sem_f).wait()
                pltpu.make_async_copy(table_hbm.at[jd_v], rd_v, sem_f).wait()

                def blend16(i, _):
                    sl = pl.ds(i * L, L)
                    wa16 = wa_v[sl]
                    wb16 = wb_v[sl]
                    wc16 = wc_v[sl]
                    wd16 = wd_v[sl]
                    for kk in range(L):
                        p = i * L + kk
                        was = _lane_bcast(wa16, kk)
                        wbs = _lane_bcast(wb16, kk)
                        wcs = _lane_bcast(wc16, kk)
                        wds = _lane_bcast(wd16, kk)
                        for c in range(C // L):
                            cs = pl.ds(c * L, L)
                            pa = was * ra_v[p, cs]
                            pb = wbs * rb_v[p, cs]
                            pc = wcs * rc_v[p, cs]
                            pd = wds * rd_v[p, cs]
                            out_v[p, cs] = ((pa + pb) + pc) + pd
                    return 0

                lax.fori_loop(0, NV, blend16, 0)

            pltpu.async_copy(out_v, out_hbm.at[pl.ds(gid * G, G)], sem_o)

        fire_xy(0, slots[0])
        fire_xy(1, slots[1])

        def pair_body(i, _):
            t0 = 2 * i
            phase1(t0, slots[0])

            @pl.when(i >= 1)
            def _():
                blend(t0 - 1, slots[1], jnp.bool_(False))

            phase1(t0 + 1, slots[1])
            blend(t0, slots[0], i == 0)
            return 0

        lax.fori_loop(0, TPW // 2, pair_body, 0)
        blend(TPW - 1, slots[1], jnp.bool_(False))
        pltpu.make_async_copy(out_v, out_hbm.at[pl.ds(0, G)], sem_o).wait()

    return k(table, xs, ys)


def kernel(input_fmap, theta):
    theta3 = theta.reshape(B, 3, 3).astype(jnp.float32)
    x = jnp.linspace(0.0, 256.0, W)
    y = jnp.linspace(0.0, 256.0, H)
    x_t, y_t = jnp.meshgrid(x, y)
    ones = jnp.ones_like(x_t.reshape(-1))
    sampling_grid = jnp.stack([x_t.reshape(-1), y_t.reshape(-1), ones])
    sampling_grid = jnp.tile(sampling_grid[None, :, :], (B, 1, 1))
    sampling_grid = sampling_grid.astype(jnp.float32)
    batch_grids = jnp.matmul(theta3, sampling_grid)
    xs = (batch_grids[:, 0, :] / batch_grids[:, 2, :]).reshape(-1)
    ys = (batch_grids[:, 1, :] / batch_grids[:, 2, :]).reshape(-1)

    table = input_fmap.reshape(N, C)
    out = _sc_sample(table, xs, ys)
    return out.reshape(B, H, W, C)


# speedup vs baseline: 6.8010x; 1.7616x over previous
import functools

import jax
import jax.numpy as jnp
from jax import lax
from jax.experimental import pallas as pl
from jax.experimental.pallas import tpu as pltpu
from jax.experimental.pallas import tpu_sc as plsc

B, H, W, C = 4, 384, 384, 96
N = B * H * W
NC, NS, L = 2, 16, 16
NW = NC * NS
G = 128
NG = N // G
TPW = NG // NW
GPB = (H * W) // G
BY, BX = 4, 16
TIERS = ((4, 16), (8, 8), (16, 4))
SBY, SBX = 2, 8
NV = G // L

_GDN = lax.GatherDimensionNumbers(
    offset_dims=(), collapsed_slice_dims=(0,), start_index_map=(0,))


def _lane_bcast(vec, kk):
    sel = jnp.full((L, 1), kk, dtype=jnp.int32)
    return lax.gather(vec, sel, _GDN, (1,),
                      mode=lax.GatherScatterMode.PROMISE_IN_BOUNDS)


def _floorclip(v):
    vc = jnp.clip(v, -1.0, 385.0)
    t = vc.astype(jnp.int32)
    tf = t.astype(jnp.float32)
    f0 = t - jnp.where(tf > vc, 1, 0)
    i0 = jnp.clip(f0, 0, W - 1)
    i1 = jnp.clip(f0 + 1, 0, W - 1)
    return i0, i1


def _sc_sample(table, xs, ys):
    mesh = plsc.VectorSubcoreMesh(core_axis_name="c", subcore_axis_name="s")

    def slot_scratch():
        return [
            pltpu.VMEM((G,), jnp.float32),
            pltpu.VMEM((G,), jnp.float32),
            pltpu.VMEM((G,), jnp.int32),
            pltpu.VMEM((G,), jnp.int32),
            pltpu.VMEM((G,), jnp.int32),
            pltpu.VMEM((G,), jnp.int32),
            pltpu.VMEM((G,), jnp.float32),
            pltpu.VMEM((G,), jnp.float32),
            pltpu.VMEM((G,), jnp.float32),
            pltpu.VMEM((G,), jnp.float32),
            pltpu.VMEM((G,), jnp.int32),
            pltpu.VMEM((G,), jnp.int32),
            pltpu.VMEM((G,), jnp.int32),
            pltpu.VMEM((G,), jnp.int32),
            pltpu.VMEM((G, C), jnp.float32),
            pltpu.VMEM((G, C), jnp.float32),
            pltpu.VMEM((G, C), jnp.float32),
            pltpu.VMEM((G, C), jnp.float32),
            pltpu.SemaphoreType.DMA,
            pltpu.SemaphoreType.DMA,
        ]

    @functools.partial(
        pl.kernel,
        mesh=mesh,
        out_type=jax.ShapeDtypeStruct((N, C), jnp.float32),
        compiler_params=pltpu.CompilerParams(use_tc_tiling_on_sc=False,
                                             needs_layout_passes=False),
        scratch_types=slot_scratch() + slot_scratch() + [
            pltpu.VMEM((G, C), jnp.float32),
            pltpu.SemaphoreType.DMA,
        ],
    )
    def k(table_hbm, xs_hbm, ys_hbm, out_hbm, *scr):
        slots = (scr[:20], scr[20:40])
        out_v, sem_o = scr[40], scr[41]
        wid = lax.axis_index("s") * NC + lax.axis_index("c")

        def gid_of(t):
            return wid + t * NW

        def fire_xy(t, s):
            xs_v, ys_v, sem_x = s[0], s[1], s[18]
            gb = gid_of(t) * G
            pltpu.async_copy(xs_hbm.at[pl.ds(gb, G)], xs_v, sem_x)
            pltpu.async_copy(ys_hbm.at[pl.ds(gb, G)], ys_v, sem_x)

        def minmax(x0_v, x1_v, y0_v, y1_v):
            xmn = x0_v[pl.ds(0, L)]
            xmx = x1_v[pl.ds(0, L)]
            ymn = y0_v[pl.ds(0, L)]
            ymx = y1_v[pl.ds(0, L)]
            for i in range(1, NV):
                sl = pl.ds(i * L, L)
                xmn = jnp.minimum(xmn, x0_v[sl])
                xmx = jnp.maximum(xmx, x1_v[sl])
                ymn = jnp.minimum(ymn, y0_v[sl])
                ymx = jnp.maximum(ymx, y1_v[sl])
            xmn = lax.reduce_min(xmn, (0,))
            xmx = lax.reduce_max(xmx, (0,))
            ymn = lax.reduce_min(ymn, (0,))
            ymx = lax.reduce_max(ymx, (0,))
            sels = []
            nofit = None
            for byt, bxt in TIERS:
                fit = jnp.logical_and(xmx - xmn < bxt, ymx - ymn < byt)
                sel = fit if nofit is None else jnp.logical_and(nofit, fit)
                nf = jnp.logical_not(fit)
                nofit = nf if nofit is None else jnp.logical_and(nofit, nf)
                sels.append(sel)
            sels.append(nofit)
            return sels, ymn, xmn

        def phase1(t, s):
            (xs_v, ys_v, x0_v, x1_v, y0_v, y1_v, wa_v, wb_v, wc_v, wd_v,
             ja_v, jb_v, jc_v, jd_v, ra_v, rb_v, rc_v, rd_v,
             sem_x, sem_f) = s
            gid = gid_of(t)
            tbase = (gid // GPB) * (H * W)
            pltpu.make_async_copy(xs_hbm.at[pl.ds(0, G)], xs_v, sem_x).wait()
            pltpu.make_async_copy(ys_hbm.at[pl.ds(0, G)], ys_v, sem_x).wait()
            res = []
            for i in range(NV):
                sl = pl.ds(i * L, L)
                x = xs_v[sl]
                y = ys_v[sl]
                x0, x1 = _floorclip(x)
                y0, y1 = _floorclip(y)
                x0f = x0.astype(jnp.float32)
                x1f = x1.astype(jnp.float32)
                y0f = y0.astype(jnp.float32)
                y1f = y1.astype(jnp.float32)
                res.append((x0, x1, y0, y1,
                            (x1f - x) * (y1f - y), (x1f - x) * (y - y0f),
                            (x - x0f) * (y1f - y), (x - x0f) * (y - y0f)))
            for i in range(NV):
                sl = pl.ds(i * L, L)
                x0, x1, y0, y1, wa, wb, wc, wd = res[i]
                x0_v[sl] = x0
                x1_v[sl] = x1
                y0_v[sl] = y0
                y1_v[sl] = y1
                wa_v[sl] = wa
                wb_v[sl] = wb
                wc_v[sl] = wc
                wd_v[sl] = wd
            @pl.when(t <= TPW - 3)
            def _():
                fire_xy(t + 2, s)

            sels, ymn, xmn = minmax(x0_v, x1_v, y0_v, y1_v)

            for vi, (byt, bxt) in enumerate(TIERS):
                @pl.when(sels[vi])
                def _(byt=byt, bxt=bxt):
                    ymn_c = jnp.minimum(ymn, H - byt)
                    xmn_c = jnp.minimum(xmn, W - bxt)
                    for i in range(NV):
                        sl = pl.ds(i * L, L)
                        dx0 = x0_v[sl] - xmn_c
                        dx1 = x1_v[sl] - xmn_c
                        ry0 = (y0_v[sl] - ymn_c) * bxt
                        ry1 = (y1_v[sl] - ymn_c) * bxt
                        ja_v[sl] = ry0 + dx0
                        jb_v[sl] = ry1 + dx0
                        jc_v[sl] = ry0 + dx1
                        jd_v[sl] = ry1 + dx1
                    rowbase = tbase + ymn_c * W + xmn_c
                    for r in range(byt):
                        pltpu.async_copy(
                            table_hbm.at[pl.ds(rowbase + r * W, bxt)],
                            rc_v.at[pl.ds(r * bxt, bxt), :], sem_f)

            @pl.when(sels[-1])
            def _():
                for i in range(NV):
                    sl = pl.ds(i * L, L)
                    x0, x1, y0, y1 = res[i][0], res[i][1], res[i][2], res[i][3]
                    sxmn = lax.reduce_min(x0, (0,))
                    sxmx = lax.reduce_max(x1, (0,))
                    symn = lax.reduce_min(y0, (0,))
                    symx = lax.reduce_max(y1, (0,))
                    sfit = jnp.logical_and(sxmx - sxmn < SBX,
                                           symx - symn < SBY)
                    symn_c = jnp.minimum(symn, H - SBY)
                    sxmn_c = jnp.minimum(sxmn, W - SBX)

                    @pl.when(sfit)
                    def _(i=i, x0=x0, x1=x1, y0=y0, y1=y1,
                          symn_c=symn_c, sxmn_c=sxmn_c):
                        sl = pl.ds(i * L, L)
                        base = i * L
                        dx0 = x0 - sxmn_c
                        dx1 = x1 - sxmn_c
                        ry0 = base + (y0 - symn_c) * SBX
                        ry1 = base + (y1 - symn_c) * SBX
                        ja_v[sl] = ry0 + dx0
                        jb_v[sl] = ry1 + dx0
                        jc_v[sl] = ry0 + dx1
                        jd_v[sl] = ry1 + dx1
                        rowbase = tbase + symn_c * W + sxmn_c
                        for r in range(SBY):
                            pltpu.async_copy(
                                table_hbm.at[pl.ds(rowbase + r * W, SBX)],
                                rc_v.at[pl.ds(base + r * SBX, SBX), :],
                                sem_f)

                    @pl.when(jnp.logical_not(sfit))
                    def _(i=i, x0=x0, x1=x1, y0=y0, y1=y1):
                        sl = pl.ds(i * L, L)
                        base = i * L
                        r0 = tbase + y0 * W
                        r1 = tbase + y1 * W
                        ja_v[sl] = r0 + x0
                        jb_v[sl] = r1 + x0
                        jc_v[sl] = r0 + x1
                        jd_v[sl] = r1 + x1
                        pltpu.async_copy(table_hbm.at[ja_v.at[sl]],
                                         ra_v.at[pl.ds(base, L), :], sem_f)
                        pltpu.async_copy(table_hbm.at[jb_v.at[sl]],
                                         rb_v.at[pl.ds(base, L), :], sem_f)
                        pltpu.async_copy(table_hbm.at[jc_v.at[sl]],
                                         rc_v.at[pl.ds(base, L), :], sem_f)
                        pltpu.async_copy(table_hbm.at[jd_v.at[sl]],
                                         rd_v.at[pl.ds(base, L), :], sem_f)

        def blend(t, s, first):
            (xs_v, ys_v, x0_v, x1_v, y0_v, y1_v, wa_v, wb_v, wc_v, wd_v,
             ja_v, jb_v, jc_v, jd_v, ra_v, rb_v, rc_v, rd_v,
             sem_x, sem_f) = s
            gid = gid_of(t)
            sels, _ymn, _xmn = minmax(x0_v, x1_v, y0_v, y1_v)
            isbox = jnp.logical_not(sels[-1])

            @pl.when(jnp.logical_not(first))
            def _():
                pltpu.make_async_copy(out_v, out_hbm.at[pl.ds(0, G)],
                                      sem_o).wait()

            cols = [jnp.arange(L, dtype=jnp.int32) + c * L
                    for c in range(C // L)]

            for vi, (byt, bxt) in enumerate(TIERS):
                @pl.when(sels[vi])
                def _(byt=byt, bxt=bxt):
                    for r in range(byt):
                        pltpu.make_async_copy(
                            table_hbm.at[pl.ds(0, bxt)],
                            rc_v.at[pl.ds(r * bxt, bxt), :], sem_f).wait()

            @pl.when(isbox)
            def _():
                def blend16(i, _):
                    sl = pl.ds(i * L, L)
                    wa16 = wa_v[sl]
                    wb16 = wb_v[sl]
                    wc16 = wc_v[sl]
                    wd16 = wd_v[sl]
                    ja16 = ja_v[sl]
                    jb16 = jb_v[sl]
                    jc16 = jc_v[sl]
                    jd16 = jd_v[sl]
                    for kk in range(L):
                        p = i * L + kk
                        was = _lane_bcast(wa16, kk)
                        wbs = _lane_bcast(wb16, kk)
                        wcs = _lane_bcast(wc16, kk)
                        wds = _lane_bcast(wd16, kk)
                        jas = _lane_bcast(ja16, kk)
                        jbs = _lane_bcast(jb16, kk)
                        jcs = _lane_bcast(jc16, kk)
                        jds = _lane_bcast(jd16, kk)
                        for c in range(C // L):
                            va = plsc.load_gather(rc_v, [jas, cols[c]])
                            vb = plsc.load_gather(rc_v, [jbs, cols[c]])
                            vc = plsc.load_gather(rc_v, [jcs, cols[c]])
                            vd = plsc.load_gather(rc_v, [jds, cols[c]])
                            pa = was * va
                            pb = wbs * vb
                            pc = wcs * vc
                            pd = wds * vd
                            out_v[p, pl.ds(c * L, L)] = ((pa + pb) + pc) + pd
                    return 0

                lax.fori_loop(0, NV, blend16, 0)

            @pl.when(sels[-1])
            def _():
                def segfit(i):
                    sl = pl.ds(i * L, L)
                    sxmn = lax.reduce_min(x0_v[sl], (0,))
                    sxmx = lax.reduce_max(x1_v[sl], (0,))
                    symn = lax.reduce_min(y0_v[sl], (0,))
                    symx = lax.reduce_max(y1_v[sl], (0,))
                    return jnp.logical_and(sxmx - sxmn < SBX,
                                           symx - symn < SBY)

                for i in range(NV):
                    sf = segfit(i)

                    @pl.when(sf)
                    def _(i=i):
                        for r in range(SBY):
                            pltpu.make_async_copy(
                                table_hbm.at[pl.ds(0, SBX)],
                                rc_v.at[pl.ds(i * L + r * SBX, SBX), :],
                                sem_f).wait()

                    @pl.when(jnp.logical_not(sf))
                    def _(i=i):
                        base = i * L
                        sl = pl.ds(base, L)
                        pltpu.make_async_copy(
                            table_hbm.at[ja_v.at[sl]],
                            ra_v.at[pl.ds(base, L), :], sem_f).wait()
                        pltpu.make_async_copy(
                            table_hbm.at[jb_v.at[sl]],
                            rb_v.at[pl.ds(base, L), :], sem_f).wait()
                        pltpu.make_async_copy(
                            table_hbm.at[jc_v.at[sl]],
                            rc_v.at[pl.ds(base, L), :], sem_f).wait()
                        pltpu.make_async_copy(
                            table_hbm.at[jd_v.at[sl]],
                            rd_v.at[pl.ds(base, L), :], sem_f).wait()

                def blend16(i, _):
                    sl = pl.ds(i * L, L)
                    wa16 = wa_v[sl]
                    wb16 = wb_v[sl]
                    wc16 = wc_v[sl]
                    wd16 = wd_v[sl]
                    sf = segfit(i)

                    @pl.when(sf)
                    def _():
                        ja16 = ja_v[sl]
                        jb16 = jb_v[sl]
                        jc16 = jc_v[sl]
                        jd16 = jd_v[sl]

                        def px(kk, _):
                            p = i * L + kk
                            was = _lane_bcast(wa16, kk)
                            wbs = _lane_bcast(wb16, kk)
                            wcs = _lane_bcast(wc16, kk)
                            wds = _lane_bcast(wd16, kk)
                            jas = _lane_bcast(ja16, kk)
                            jbs = _lane_bcast(jb16, kk)
                            jcs = _lane_bcast(jc16, kk)
                            jds = _lane_bcast(jd16, kk)
                            for c in range(C // L):
                                va = plsc.load_gather(rc_v, [jas, cols[c]])
                                vb = plsc.load_gather(rc_v, [jbs, cols[c]])
                                vc = plsc.load_gather(rc_v, [jcs, cols[c]])
                                vd = plsc.load_gather(rc_v, [jds, cols[c]])
                                pa = was * va
                                pb = wbs * vb
                                pc = wcs * vc
                                pd = wds * vd
                                out_v[p, pl.ds(c * L, L)] = (
                                    ((pa + pb) + pc) + pd)
                            return 0

                        lax.fori_loop(0, L, px, 0)

                    @pl.when(jnp.logical_not(sf))
                    def _():
                        def px(kk, _):
                            p = i * L + kk
                            was = _lane_bcast(wa16, kk)
                            wbs = _lane_bcast(wb16, kk)
                            wcs = _lane_bcast(wc16, kk)
                            wds = _lane_bcast(wd16, kk)
                            for c in range(C // L):
                                cs = pl.ds(c * L, L)
                                pa = was * ra_v[p, cs]
                                pb = wbs * rb_v[p, cs]
                                pc = wcs * rc_v[p, cs]
                                pd = wds * rd_v[p, cs]
                                out_v[p, cs] = ((pa + pb) + pc) + pd
                            return 0

                        lax.fori_loop(0, L, px, 0)
                    return 0

                lax.fori_loop(0, NV, blend16, 0)

            pltpu.async_copy(out_v, out_hbm.at[pl.ds(gid * G, G)], sem_o)

        fire_xy(0, slots[0])
        fire_xy(1, slots[1])

        def pair_body(i, _):
            t0 = 2 * i
            phase1(t0, slots[0])

            @pl.when(i >= 1)
            def _():
                blend(t0 - 1, slots[1], jnp.bool_(False))

            phase1(t0 + 1, slots[1])
            blend(t0, slots[0], i == 0)
            return 0

        lax.fori_loop(0, TPW // 2, pair_body, 0)
        blend(TPW - 1, slots[1], jnp.bool_(False))
        pltpu.make_async_copy(out_v, out_hbm.at[pl.ds(0, G)], sem_o).wait()

    return k(table, xs, ys)


def kernel(input_fmap, theta):
    theta3 = theta.reshape(B, 3, 3).astype(jnp.float32)
    x = jnp.linspace(0.0, 256.0, W)
    y = jnp.linspace(0.0, 256.0, H)
    x_t, y_t = jnp.meshgrid(x, y)
    ones = jnp.ones_like(x_t.reshape(-1))
    sampling_grid = jnp.stack([x_t.reshape(-1), y_t.reshape(-1), ones])
    sampling_grid = jnp.tile(sampling_grid[None, :, :], (B, 1, 1))
    sampling_grid = sampling_grid.astype(jnp.float32)
    batch_grids = jnp.matmul(theta3, sampling_grid)
    xs = (batch_grids[:, 0, :] / batch_grids[:, 2, :]).reshape(-1)
    ys = (batch_grids[:, 1, :] / batch_grids[:, 2, :]).reshape(-1)

    table = input_fmap.reshape(N, C)
    out = _sc_sample(table, xs, ys)
    return out.reshape(B, H, W, C)
